# baseline (speedup 1.0000x reference)
import functools

import jax
import jax.numpy as jnp
from jax import lax
from jax.experimental import pallas as pl
from jax.experimental.pallas import tpu as pltpu
from jax.experimental.pallas import tpu_sc as plsc

_VOCAB = 1000
_VPAD = 1024
_B, _L = 1024, 50
_N = _B * _L

_info = plsc.get_sparse_core_info()
_NC = _info.num_cores
_NS = _info.num_subcores
_NW = _NC * _NS
_ROWS_PW = _B // _NW
_LPAD = 56
_IPW = _ROWS_PW * _LPAD

_mesh = plsc.VectorSubcoreMesh(core_axis_name="c", subcore_axis_name="s")


@functools.partial(
    pl.kernel,
    mesh=_mesh,
    out_type=jax.ShapeDtypeStruct((_B, _LPAD, _VPAD), jnp.float32),
    scratch_types=[
        pltpu.VMEM((_IPW,), jnp.int32),
        [pltpu.VMEM((_LPAD, _VPAD), jnp.float32)] * 2,
        [pltpu.SemaphoreType.DMA] * 2,
        [pltpu.SemaphoreType.DMA] * 2,
    ],
)
def _gather_rows(idx_hbm, table_hbm, out_hbm, idx_v, bufs, semg, semw):
    wid = lax.axis_index("s") * _NC + lax.axis_index("c")
    base = wid * _IPW
    b0 = wid * _ROWS_PW
    pltpu.sync_copy(idx_hbm.at[pl.ds(base, _IPW)], idx_v)

    def gather(b, buf, sem):
        return pltpu.make_async_copy(
            table_hbm.at[idx_v.at[pl.ds(b * _LPAD, _LPAD)]], buf, sem)

    def write(b, buf, sem):
        return pltpu.make_async_copy(buf, out_hbm.at[b0 + b], sem)

    gather(0, bufs[0], semg[0]).start()

    def body(b, carry):
        for p in (0, 1):
            gather(b + p, bufs[p], semg[p]).wait()

            @pl.when(b + p >= 1)
            def _():
                write(b + p - 1, bufs[1 - p], semw[1 - p]).wait()

            @pl.when(b + p + 1 < _ROWS_PW)
            def _():
                gather(b + p + 1, bufs[1 - p], semg[1 - p]).start()

            write(b + p, bufs[p], semw[p]).start()
        return carry

    lax.fori_loop(0, _ROWS_PW // 2, lambda i, c: body(i * 2, c), 0)
    write(_ROWS_PW - 1, bufs[1], semw[1]).wait()


_BB = 4


def _slice_body(i_ref, o_ref):
    o_ref[...] = i_ref[:, : _L, : _VOCAB]


_slice_lanes = pl.pallas_call(
    _slice_body,
    grid=(_B // _BB,),
    in_specs=[pl.BlockSpec((_BB, _LPAD, _VPAD), lambda b: (b, 0, 0))],
    out_specs=pl.BlockSpec((_BB, _L, _VOCAB), lambda b: (b, 0, 0)),
    out_shape=jax.ShapeDtypeStruct((_B, _L, _VOCAB), jnp.float32),
)


def kernel(idx, logits):
    idx_p = jnp.pad(idx.astype(jnp.int32), ((0, 0), (0, _LPAD - _L)))
    flat = idx_p.reshape(_B * _LPAD)
    table = jnp.pad(logits, ((0, 0), (0, _VPAD - _VOCAB)))
    padded = _gather_rows(flat, table)
    return padded[:, : _L, : _VOCAB]
